# baseline (speedup 1.0000x reference)
import functools

import jax
import jax.numpy as jnp
from jax import lax
from jax.experimental import pallas as pl
from jax.experimental.pallas import tpu as pltpu
from jax.experimental.pallas import tpu_sc as plsc

_info = plsc.get_sparse_core_info()
_NC = _info.num_cores
_NS = _info.num_subcores
_NW = _NC * _NS

_GROUP = 128


@functools.partial(jax.jit, static_argnames=("n_rows", "dim"))
def _sc_gather(weight, idx, *, n_rows, dim):
    b_per_w = n_rows // _NW
    n_groups = b_per_w // _GROUP
    mesh = plsc.VectorSubcoreMesh(core_axis_name="c", subcore_axis_name="s")

    @functools.partial(
        pl.kernel,
        mesh=mesh,
        out_type=jax.ShapeDtypeStruct((n_rows, dim), jnp.float32),
        scratch_types=[
            pltpu.VMEM((b_per_w,), jnp.int32),
            pltpu.VMEM((2, _GROUP, dim), jnp.float32),
            pltpu.SemaphoreType.DMA((2,)),
            pltpu.SemaphoreType.DMA((2,)),
        ],
        compiler_params=pltpu.CompilerParams(use_tc_tiling_on_sc=False),
    )
    def k(table_hbm, idx_hbm, out_hbm, idx_v, rows_v, gsem, ssem):
        wid = lax.axis_index("s") * _NC + lax.axis_index("c")
        base = wid * b_per_w
        pltpu.sync_copy(idx_hbm.at[pl.ds(base, b_per_w)], idx_v)

        def gather_group(grp, p):
            for j in range(_GROUP // 16):
                off = pl.multiple_of(grp * _GROUP + j * 16, 8)
                iv = idx_v[pl.ds(off, 16)]
                pltpu.make_async_copy(
                    table_hbm.at[iv],
                    rows_v.at[p, pl.ds(j * 16, 16), :],
                    gsem.at[p],
                ).start()

        def wait_gather_group(grp, p):
            for j in range(_GROUP // 16):
                off = pl.multiple_of(grp * _GROUP + j * 16, 8)
                iv = idx_v[pl.ds(off, 16)]
                pltpu.make_async_copy(
                    table_hbm.at[iv],
                    rows_v.at[p, pl.ds(j * 16, 16), :],
                    gsem.at[p],
                ).wait()

        def store_copy(grp, p):
            off = pl.multiple_of(grp * _GROUP, 8)
            return pltpu.make_async_copy(
                rows_v.at[p],
                out_hbm.at[pl.ds(base + off, _GROUP), :],
                ssem.at[p],
            )

        gather_group(0, 0)

        def outer(o, carry):
            for p in (0, 1):
                g = 2 * o + p
                q = 1 - p

                @pl.when(g >= 1)
                def _():
                    store_copy(g - 1, q).wait()

                @pl.when(g + 1 < n_groups)
                def _():
                    gather_group(g + 1, q)

                wait_gather_group(g, p)
                store_copy(g, p).start()
            return carry

        lax.fori_loop(0, n_groups // 2, outer, 0)
        store_copy(n_groups - 1, 1).wait()

    return k(weight, idx)


def kernel(input, weight):
    n_rows = input.size
    dim = weight.shape[1]
    idx = input.reshape(n_rows).astype(jnp.int32)
    out = _sc_gather(weight, idx, n_rows=n_rows, dim=dim)
    return out.reshape(input.shape + (dim,))


# speedup vs baseline: 1.0356x; 1.0356x over previous
import functools

import jax
import jax.numpy as jnp
from jax import lax
from jax.experimental import pallas as pl
from jax.experimental.pallas import tpu as pltpu
from jax.experimental.pallas import tpu_sc as plsc

_info = plsc.get_sparse_core_info()
_NC = _info.num_cores
_NS = _info.num_subcores
_NW = _NC * _NS

_CHUNK = 128
_NBUF = 8
_K = 4


@functools.partial(jax.jit, static_argnames=("n_rows", "dim"))
def _sc_gather(weight, idx, *, n_rows, dim):
    b_per_w = n_rows // _NW
    n_chunks = b_per_w // _CHUNK
    n_groups = n_chunks // _NBUF
    mesh = plsc.VectorSubcoreMesh(core_axis_name="c", subcore_axis_name="s")

    @functools.partial(
        pl.kernel,
        mesh=mesh,
        out_type=jax.ShapeDtypeStruct((n_rows, dim), jnp.float32),
        scratch_types=[
            pltpu.VMEM((b_per_w,), jnp.int32),
            pltpu.VMEM((_NBUF, _CHUNK, dim), jnp.float32),
            pltpu.SemaphoreType.DMA((_NBUF,)),
            pltpu.SemaphoreType.DMA((_NBUF,)),
        ],
        compiler_params=pltpu.CompilerParams(use_tc_tiling_on_sc=False),
    )
    def k(table_hbm, idx_hbm, out_hbm, idx_v, rows_v, gsem, ssem):
        wid = lax.axis_index("s") * _NC + lax.axis_index("c")
        base = wid * b_per_w
        pltpu.sync_copy(idx_hbm.at[pl.ds(base, b_per_w)], idx_v)

        def gather_copy(chunk, b):
            off = pl.multiple_of(chunk * _CHUNK, 8)
            return pltpu.make_async_copy(
                table_hbm.at[idx_v.at[pl.ds(off, _CHUNK)]],
                rows_v.at[b],
                gsem.at[b],
            )

        def store_copy(chunk, b):
            off = pl.multiple_of(chunk * _CHUNK, 8)
            return pltpu.make_async_copy(
                rows_v.at[b],
                out_hbm.at[pl.ds(base + off, _CHUNK), :],
                ssem.at[b],
            )

        for j in range(_K):
            gather_copy(j, j).start()

        def group(grp, carry):
            for j in range(_NBUF):
                g = grp * _NBUF + j
                b = j
                b2 = (j + _K) % _NBUF
                gather_copy(g, b).wait()
                store_copy(g, b).start()

                @pl.when(g >= _K)
                def _():
                    store_copy(g - _K, b2).wait()

                @pl.when(g + _K < n_chunks)
                def _():
                    gather_copy(g + _K, b2).start()

            return carry

        lax.fori_loop(0, n_groups, group, 0)
        for j in range(_K):
            g = n_chunks - _K + j
            store_copy(g, g % _NBUF).wait()

    return k(weight, idx)


def kernel(input, weight):
    n_rows = input.size
    dim = weight.shape[1]
    idx = input.reshape(n_rows).astype(jnp.int32)
    out = _sc_gather(weight, idx, n_rows=n_rows, dim=dim)
    return out.reshape(input.shape + (dim,))


# speedup vs baseline: 1.0377x; 1.0020x over previous
import functools

import jax
import jax.numpy as jnp
from jax import lax
from jax.experimental import pallas as pl
from jax.experimental.pallas import tpu as pltpu
from jax.experimental.pallas import tpu_sc as plsc

_info = plsc.get_sparse_core_info()
_NC = _info.num_cores
_NS = _info.num_subcores
_NW = _NC * _NS

_CHUNK = 128
_NBUF = 8


@functools.partial(jax.jit, static_argnames=("n_rows", "dim"))
def _sc_gather(weight, idx, *, n_rows, dim):
    b_per_w = n_rows // _NW
    n_chunks = b_per_w // _CHUNK
    n_groups = n_chunks // _NBUF
    mesh = plsc.VectorSubcoreMesh(core_axis_name="c", subcore_axis_name="s")

    @functools.partial(
        pl.kernel,
        mesh=mesh,
        out_type=jax.ShapeDtypeStruct((n_rows, dim), jnp.float32),
        scratch_types=[
            pltpu.VMEM((b_per_w,), jnp.int32),
            pltpu.VMEM((_NBUF, _CHUNK, dim), jnp.float32),
            pltpu.SemaphoreType.DMA((_NBUF,)),
            pltpu.SemaphoreType.DMA((_NBUF,)),
        ],
        compiler_params=pltpu.CompilerParams(use_tc_tiling_on_sc=False),
    )
    def k(table_hbm, idx_hbm, out_hbm, idx_v, rows_v, gsem, ssem):
        wid = lax.axis_index("s") * _NC + lax.axis_index("c")
        base = wid * b_per_w
        pltpu.sync_copy(idx_hbm.at[pl.ds(base, b_per_w)], idx_v)

        def gather_copy(chunk, b):
            off = pl.multiple_of(chunk * _CHUNK, 8)
            return pltpu.make_async_copy(
                table_hbm.at[idx_v.at[pl.ds(off, _CHUNK)]],
                rows_v.at[b],
                gsem.at[b],
            )

        def store_copy(chunk, b):
            off = pl.multiple_of(chunk * _CHUNK, 8)
            return pltpu.make_async_copy(
                rows_v.at[b],
                out_hbm.at[pl.ds(base + off, _CHUNK), :],
                ssem.at[b],
            )

        for b in range(_NBUF):
            gather_copy(b, b).start()

        def group(grp, carry):
            for b in range(_NBUF):
                chunk = grp * _NBUF + b
                gather_copy(chunk, b).wait()
                store_copy(chunk, b).start()
                store_copy(chunk, b).wait()

                @pl.when(grp < n_groups - 1)
                def _():
                    gather_copy(chunk + _NBUF, b).start()

            return carry

        lax.fori_loop(0, n_groups, group, 0)

    return k(weight, idx)


def kernel(input, weight):
    n_rows = input.size
    dim = weight.shape[1]
    idx = input.reshape(n_rows).astype(jnp.int32)
    out = _sc_gather(weight, idx, n_rows=n_rows, dim=dim)
    return out.reshape(input.shape + (dim,))
